# Initial kernel scaffold; baseline (speedup 1.0000x reference)
import functools

import jax
import jax.numpy as jnp
from jax import lax
from jax.experimental import pallas as pl
from jax.experimental.pallas import tpu as pltpu
from jax.experimental.pallas import tpu_sc as plsc

D_MODEL = 64
NUM_INDICES = 16384 * 50

NC = 2
NS = 16
NW = NC * NS

IDX_MINOR = 128
PER_W = NUM_INDICES // NW
CHUNK = 512
K = CHUNK // IDX_MINOR
NCHUNK = PER_W // CHUNK
IDX_ROWS = PER_W // IDX_MINOR


def _body(idx_hbm, table_hbm, out_hbm, idx_v, rows_v, gsem):
    cid = lax.axis_index("c")
    sid = lax.axis_index("s")
    wid = sid * NC + cid

    pltpu.sync_copy(idx_hbm.at[wid], idx_v)

    def chunk_body(c, carry):
        base = wid * PER_W + c * CHUNK
        for j in range(K):
            pltpu.async_copy(
                table_hbm.at[idx_v.at[c * K + j]],
                rows_v.at[pl.ds(j * IDX_MINOR, IDX_MINOR)],
                gsem,
            )
        for j in range(K):
            pltpu.make_async_copy(
                table_hbm.at[idx_v.at[0]],
                rows_v.at[pl.ds(0, IDX_MINOR)],
                gsem,
            ).wait()
        pltpu.sync_copy(rows_v, out_hbm.at[pl.ds(base, CHUNK)])
        return carry

    lax.fori_loop(0, NCHUNK, chunk_body, 0)


@jax.jit
def _gather(idx, table):
    mesh = plsc.VectorSubcoreMesh(core_axis_name="c", subcore_axis_name="s")
    f = pl.kernel(
        _body,
        out_type=jax.ShapeDtypeStruct((NUM_INDICES, D_MODEL), jnp.float32),
        mesh=mesh,
        scratch_types=[
            pltpu.VMEM((IDX_ROWS, IDX_MINOR), jnp.int32),
            pltpu.VMEM((CHUNK, D_MODEL), jnp.float32),
            pltpu.SemaphoreType.DMA,
        ],
    )
    return f(idx, table)


def kernel(type_input, segment_embeddings_weight):
    batch, hist = type_input.shape
    idx = type_input.reshape(NW, IDX_ROWS, IDX_MINOR).astype(jnp.int32)
    out = _gather(idx, segment_embeddings_weight)
    return out.reshape(batch, hist, D_MODEL)


# speedup vs baseline: 6.0026x; 6.0026x over previous
import functools

import jax
import jax.numpy as jnp
from jax import lax
from jax.experimental import pallas as pl
from jax.experimental.pallas import tpu as pltpu
from jax.experimental.pallas import tpu_sc as plsc

D_MODEL = 64
NUM_INDICES = 16384 * 50

NC = 2
NS = 16
NW = NC * NS

IDX_MINOR = 128
PER_W = NUM_INDICES // NW
CHUNK = 512
K = CHUNK // IDX_MINOR
NCHUNK = PER_W // CHUNK
IDX_ROWS = PER_W // IDX_MINOR


def _body(idx_hbm, table_hbm, out_hbm, idx_v, rows_v, gsem):
    cid = lax.axis_index("c")
    sid = lax.axis_index("s")
    wid = sid * NC + cid

    pltpu.sync_copy(idx_hbm.at[wid], idx_v)

    def chunk_body(c, carry):
        base = wid * PER_W + c * CHUNK
        for j in range(K):
            pltpu.async_copy(
                table_hbm.at[idx_v.at[c * K + j]],
                rows_v.at[pl.ds(j * IDX_MINOR, IDX_MINOR)],
                gsem,
            )
        for j in range(K):
            pltpu.make_async_copy(
                table_hbm.at[idx_v.at[0]],
                rows_v.at[pl.ds(0, IDX_MINOR)],
                gsem,
            ).wait()
        pltpu.sync_copy(rows_v, out_hbm.at[pl.ds(base, CHUNK)])
        return carry

    lax.fori_loop(0, NCHUNK, chunk_body, 0)


@jax.jit
def _gather(idx, table):
    mesh = plsc.VectorSubcoreMesh(core_axis_name="c", subcore_axis_name="s")
    f = pl.kernel(
        _body,
        out_type=jax.ShapeDtypeStruct((NUM_INDICES, D_MODEL), jnp.float32),
        mesh=mesh,
        scratch_types=[
            pltpu.VMEM((IDX_ROWS, IDX_MINOR), jnp.int32),
            pltpu.VMEM((CHUNK, D_MODEL), jnp.float32),
            pltpu.SemaphoreType.DMA,
        ],
        compiler_params=pltpu.CompilerParams(use_tc_tiling_on_sc=False),
    )
    return f(idx, table)


def kernel(type_input, segment_embeddings_weight):
    batch, hist = type_input.shape
    idx = type_input.reshape(NW, IDX_ROWS, IDX_MINOR).astype(jnp.int32)
    out = _gather(idx, segment_embeddings_weight)
    return out.reshape(batch, hist, D_MODEL)


# speedup vs baseline: 6.2556x; 1.0421x over previous
import functools

import jax
import jax.numpy as jnp
from jax import lax
from jax.experimental import pallas as pl
from jax.experimental.pallas import tpu as pltpu
from jax.experimental.pallas import tpu_sc as plsc

D_MODEL = 64
NUM_INDICES = 16384 * 50

NC = 2
NS = 16
NW = NC * NS

IDX_MINOR = 128
PER_W = NUM_INDICES // NW
CHUNK = 512
K = CHUNK // IDX_MINOR
NCHUNK = PER_W // CHUNK
IDX_ROWS = PER_W // IDX_MINOR


def _body(idx_hbm, table_hbm, out_hbm, idx_v, rows0, rows1, g0, g1, o0, o1):
    cid = lax.axis_index("c")
    sid = lax.axis_index("s")
    wid = sid * NC + cid
    rows = (rows0, rows1)
    gsem = (g0, g1)
    osem = (o0, o1)

    pltpu.sync_copy(idx_hbm.at[wid], idx_v)

    def start_gathers(c, s):
        for j in range(K):
            pltpu.async_copy(
                table_hbm.at[idx_v.at[c * K + j]],
                rows[s].at[pl.ds(j * IDX_MINOR, IDX_MINOR)],
                gsem[s],
            )

    def wait_gathers(s):
        for _ in range(K):
            pltpu.make_async_copy(
                table_hbm.at[pl.ds(0, IDX_MINOR)],
                rows[s].at[pl.ds(0, IDX_MINOR)],
                gsem[s],
            ).wait()

    def start_out(c, s):
        pltpu.async_copy(
            rows[s], out_hbm.at[pl.ds(wid * PER_W + c * CHUNK, CHUNK)], osem[s]
        )

    def wait_out(s):
        pltpu.make_async_copy(
            rows[s], out_hbm.at[pl.ds(0, CHUNK)], osem[s]
        ).wait()

    start_gathers(0, 0)
    start_gathers(1, 1)
    wait_gathers(0)
    start_out(0, 0)

    def pair_body(i, carry):
        g = 1 + 2 * i
        for b in range(2):
            c = g + b
            s = (1 - b)
            o = 1 - s
            wait_out(o)
            start_gathers(c + 1, o)
            wait_gathers(s)
            start_out(c, s)
        return carry

    lax.fori_loop(0, (NCHUNK - 2) // 2, pair_body, 0)

    wait_out(0)
    wait_gathers(1)
    start_out(NCHUNK - 1, 1)
    wait_out(1)


@jax.jit
def _gather(idx, table):
    mesh = plsc.VectorSubcoreMesh(core_axis_name="c", subcore_axis_name="s")
    f = pl.kernel(
        _body,
        out_type=jax.ShapeDtypeStruct((NUM_INDICES, D_MODEL), jnp.float32),
        mesh=mesh,
        scratch_types=[
            pltpu.VMEM((IDX_ROWS, IDX_MINOR), jnp.int32),
            pltpu.VMEM((CHUNK, D_MODEL), jnp.float32),
            pltpu.VMEM((CHUNK, D_MODEL), jnp.float32),
            pltpu.SemaphoreType.DMA,
            pltpu.SemaphoreType.DMA,
            pltpu.SemaphoreType.DMA,
            pltpu.SemaphoreType.DMA,
        ],
        compiler_params=pltpu.CompilerParams(use_tc_tiling_on_sc=False),
    )
    return f(idx, table)


def kernel(type_input, segment_embeddings_weight):
    batch, hist = type_input.shape
    idx = type_input.reshape(NW, IDX_ROWS, IDX_MINOR).astype(jnp.int32)
    out = _gather(idx, segment_embeddings_weight)
    return out.reshape(batch, hist, D_MODEL)
